# baseline (speedup 1.0000x reference)
import functools

import jax
import jax.numpy as jnp
from jax import lax
from jax.experimental import pallas as pl
from jax.experimental.pallas import tpu as pltpu
from jax.experimental.pallas import tpu_sc as plsc

VOCAB = 100000
EMBED = 768
CTX = 2048
B = 4
S = 2048

NUM_CORES = 2
NUM_SUBCORES = 16
NUM_WORKERS = NUM_CORES * NUM_SUBCORES
S_BLK = S // NUM_WORKERS
S_CHUNK = 8
NCHUNK = S_BLK // S_CHUNK
ROWS = B * S_CHUNK
NBUF = 3
LANES = 16
COL_CHUNKS = EMBED // LANES


def _emb_kernel(idx_hbm, tok_hbm, pos_hbm, out_hbm, idx_v, pbuf, rbuf, gsems,
                psems, wsems):
    wid = lax.axis_index("s") * NUM_CORES + lax.axis_index("c")
    s0 = wid * S_BLK

    for b in range(B):
        pltpu.sync_copy(idx_hbm.at[b, pl.ds(s0, S_BLK)], idx_v.at[b])

    def start_gather(c):
        gs = []
        for b in range(B):
            idx_slice = idx_v.at[b, pl.ds(c * S_CHUNK, S_CHUNK)]
            gs.append(
                pltpu.async_copy(
                    tok_hbm.at[idx_slice],
                    rbuf.at[c % NBUF, pl.ds(b * S_CHUNK, S_CHUNK)],
                    gsems.at[c % NBUF]))
        return gs

    def start_posload(c):
        return pltpu.async_copy(pos_hbm.at[pl.ds(s0 + c * S_CHUNK, S_CHUNK)],
                                pbuf.at[c % NBUF], psems.at[c % NBUF])

    gathers = {0: start_gather(0)}
    posloads = {0: start_posload(0)}
    writes = {}
    for c in range(NCHUNK):
        if c >= 2:
            for w in writes[c - 2]:
                w.wait()
        if c + 1 < NCHUNK:
            gathers[c + 1] = start_gather(c + 1)
            posloads[c + 1] = start_posload(c + 1)
        for g in gathers[c]:
            g.wait()
        posloads[c].wait()

        buf = rbuf.at[c % NBUF]
        pos = pbuf.at[c % NBUF]

        def s_body(t, carry):
            ps = []
            for j in range(COL_CHUNKS):
                ps.append(pos[t, pl.ds(j * LANES, LANES)])
            for b in range(B):
                r = b * S_CHUNK + t
                for j in range(COL_CHUNKS):
                    sl = pl.ds(j * LANES, LANES)
                    buf[r, sl] = buf[r, sl] + ps[j]
            return carry

        lax.fori_loop(0, S_CHUNK, s_body, 0)

        ws = []
        for b in range(B):
            base = b * S + s0 + c * S_CHUNK
            ws.append(
                pltpu.async_copy(buf.at[pl.ds(b * S_CHUNK, S_CHUNK)],
                                 out_hbm.at[pl.ds(base, S_CHUNK)],
                                 wsems.at[c % NBUF]))
        writes[c] = ws
    for c in (NCHUNK - 2, NCHUNK - 1):
        for w in writes[c]:
            w.wait()


@jax.jit
def _run(idx2d, token_table, pos_table):
    mesh = plsc.VectorSubcoreMesh(core_axis_name="c", subcore_axis_name="s")
    f = functools.partial(
        pl.kernel,
        mesh=mesh,
        out_type=jax.ShapeDtypeStruct((B * S, EMBED), jnp.float32),
        scratch_types=[
            pltpu.VMEM((B, S_BLK), jnp.int32),
            pltpu.VMEM((NBUF, S_CHUNK, EMBED), jnp.float32),
            pltpu.VMEM((NBUF, ROWS, EMBED), jnp.float32),
            pltpu.SemaphoreType.DMA((NBUF,)),
            pltpu.SemaphoreType.DMA((NBUF,)),
            pltpu.SemaphoreType.DMA((NBUF,)),
        ],
    )(_emb_kernel)
    return f(idx2d, token_table, pos_table)


def kernel(tok_idx, token_table, pos_table):
    out = _run(tok_idx.astype(jnp.int32), token_table, pos_table)
    return out.reshape(B, S, EMBED)


# speedup vs baseline: 1.0227x; 1.0227x over previous
import functools

import jax
import jax.numpy as jnp
from jax import lax
from jax.experimental import pallas as pl
from jax.experimental.pallas import tpu as pltpu
from jax.experimental.pallas import tpu_sc as plsc

VOCAB = 100000
EMBED = 768
CTX = 2048
B = 4
S = 2048

NUM_CORES = 2
NUM_SUBCORES = 16
NUM_WORKERS = NUM_CORES * NUM_SUBCORES
S_BLK = S // NUM_WORKERS
S_CHUNK = 8
NCHUNK = S_BLK // S_CHUNK
ROWS = B * S_CHUNK
NBUF = 4
LANES = 16
COL_CHUNKS = EMBED // LANES
HALF = COL_CHUNKS // 2


def _emb_kernel(idx_hbm, tok_hbm, pos_hbm, out_hbm, idx_v, pbuf, rbuf, gsems,
                psems, wsems):
    wid = lax.axis_index("s") * NUM_CORES + lax.axis_index("c")
    s0 = wid * S_BLK

    pltpu.sync_copy(idx_hbm.at[wid], idx_v)

    def start_gather(c):
        return pltpu.async_copy(tok_hbm.at[idx_v.at[c]], rbuf.at[c % NBUF],
                                gsems.at[c % NBUF])

    def start_posload(c):
        return pltpu.async_copy(pos_hbm.at[pl.ds(s0 + c * S_CHUNK, S_CHUNK)],
                                pbuf.at[c % NBUF], psems.at[c % NBUF])

    gathers = {0: start_gather(0)}
    posloads = {0: start_posload(0)}
    writes = {}
    for c in range(NCHUNK):
        if c >= 3:
            for w in writes[c - 3]:
                w.wait()
        if c + 1 < NCHUNK:
            gathers[c + 1] = start_gather(c + 1)
            posloads[c + 1] = start_posload(c + 1)
        gathers[c].wait()
        posloads[c].wait()

        buf = rbuf.at[c % NBUF]
        pos = pbuf.at[c % NBUF]

        def s_body(t, carry):
            for half in range(2):
                j0 = half * HALF
                ps = []
                for j in range(j0, j0 + HALF):
                    ps.append(pos[t, pl.ds(j * LANES, LANES)])
                for b in range(B):
                    r = b * S_CHUNK + t
                    for j in range(j0, j0 + HALF):
                        sl = pl.ds(j * LANES, LANES)
                        buf[r, sl] = buf[r, sl] + ps[j - j0]
            return carry

        lax.fori_loop(0, S_CHUNK, s_body, 0)

        ws = []
        for b in range(B):
            base = b * S + s0 + c * S_CHUNK
            ws.append(
                pltpu.async_copy(buf.at[pl.ds(b * S_CHUNK, S_CHUNK)],
                                 out_hbm.at[pl.ds(base, S_CHUNK)],
                                 wsems.at[c % NBUF]))
        writes[c] = ws
    for c in (NCHUNK - 3, NCHUNK - 2, NCHUNK - 1):
        for w in writes[c]:
            w.wait()


@jax.jit
def _run(idx_re, token_table, pos_table):
    mesh = plsc.VectorSubcoreMesh(core_axis_name="c", subcore_axis_name="s")
    f = functools.partial(
        pl.kernel,
        mesh=mesh,
        out_type=jax.ShapeDtypeStruct((B * S, EMBED), jnp.float32),
        scratch_types=[
            pltpu.VMEM((NCHUNK, ROWS), jnp.int32),
            pltpu.VMEM((NBUF, S_CHUNK, EMBED), jnp.float32),
            pltpu.VMEM((NBUF, ROWS, EMBED), jnp.float32),
            pltpu.SemaphoreType.DMA((NBUF,)),
            pltpu.SemaphoreType.DMA((NBUF,)),
            pltpu.SemaphoreType.DMA((NBUF,)),
        ],
    )(_emb_kernel)
    return f(idx_re, token_table, pos_table)


def kernel(tok_idx, token_table, pos_table):
    idx_re = jnp.transpose(
        tok_idx.astype(jnp.int32).reshape(B, NUM_WORKERS, NCHUNK, S_CHUNK),
        (1, 2, 0, 3)).reshape(NUM_WORKERS, NCHUNK, ROWS)
    out = _run(idx_re, token_table, pos_table)
    return out.reshape(B, S, EMBED)


# speedup vs baseline: 1.0279x; 1.0050x over previous
import functools

import jax
import jax.numpy as jnp
from jax import lax
from jax.experimental import pallas as pl
from jax.experimental.pallas import tpu as pltpu
from jax.experimental.pallas import tpu_sc as plsc

VOCAB = 100000
EMBED = 768
CTX = 2048
B = 4
S = 2048

NUM_CORES = 2
NUM_SUBCORES = 16
NUM_WORKERS = NUM_CORES * NUM_SUBCORES
S_BLK = S // NUM_WORKERS
S_CHUNK = 8
NCHUNK = S_BLK // S_CHUNK
ROWS = B * S_CHUNK
NBUF = 3
LANES = 16
COL_CHUNKS = EMBED // LANES
HALF = COL_CHUNKS


def _emb_kernel(idx_hbm, tok_hbm, pos_hbm, out_hbm, idx_v, pbuf, rbuf, gsems,
                psems, wsems):
    wid = lax.axis_index("s") * NUM_CORES + lax.axis_index("c")
    s0 = wid * S_BLK

    pltpu.sync_copy(idx_hbm.at[wid], idx_v)

    def start_gather(c):
        return pltpu.async_copy(tok_hbm.at[idx_v.at[c]], rbuf.at[c % NBUF],
                                gsems.at[c % NBUF])

    def start_posload(c):
        return pltpu.async_copy(pos_hbm.at[pl.ds(s0 + c * S_CHUNK, S_CHUNK)],
                                pbuf.at[c % NBUF], psems.at[c % NBUF])

    gathers = {0: start_gather(0)}
    posloads = {0: start_posload(0)}
    writes = {}
    for c in range(NCHUNK):
        if c >= 2:
            for w in writes[c - 2]:
                w.wait()
        if c + 1 < NCHUNK:
            gathers[c + 1] = start_gather(c + 1)
            posloads[c + 1] = start_posload(c + 1)
        gathers[c].wait()
        posloads[c].wait()

        buf = rbuf.at[c % NBUF]
        pos = pbuf.at[c % NBUF]

        def s_body(t, carry):
            for half in range(COL_CHUNKS // HALF):
                j0 = half * HALF
                ps = []
                for j in range(j0, j0 + HALF):
                    ps.append(pos[t, pl.ds(j * LANES, LANES)])
                for b in range(B):
                    r = b * S_CHUNK + t
                    for j in range(j0, j0 + HALF):
                        sl = pl.ds(j * LANES, LANES)
                        buf[r, sl] = buf[r, sl] + ps[j - j0]
            return carry

        lax.fori_loop(0, S_CHUNK, s_body, 0)

        ws = []
        for b in range(B):
            base = b * S + s0 + c * S_CHUNK
            ws.append(
                pltpu.async_copy(buf.at[pl.ds(b * S_CHUNK, S_CHUNK)],
                                 out_hbm.at[pl.ds(base, S_CHUNK)],
                                 wsems.at[c % NBUF]))
        writes[c] = ws
    for c in (NCHUNK - 2, NCHUNK - 1):
        for w in writes[c]:
            w.wait()


@jax.jit
def _run(idx_re, token_table, pos_table):
    mesh = plsc.VectorSubcoreMesh(core_axis_name="c", subcore_axis_name="s")
    f = functools.partial(
        pl.kernel,
        mesh=mesh,
        out_type=jax.ShapeDtypeStruct((B * S, EMBED), jnp.float32),
        scratch_types=[
            pltpu.VMEM((NCHUNK, ROWS), jnp.int32),
            pltpu.VMEM((NBUF, S_CHUNK, EMBED), jnp.float32),
            pltpu.VMEM((NBUF, ROWS, EMBED), jnp.float32),
            pltpu.SemaphoreType.DMA((NBUF,)),
            pltpu.SemaphoreType.DMA((NBUF,)),
            pltpu.SemaphoreType.DMA((NBUF,)),
        ],
    )(_emb_kernel)
    return f(idx_re, token_table, pos_table)


def kernel(tok_idx, token_table, pos_table):
    idx_re = jnp.transpose(
        tok_idx.astype(jnp.int32).reshape(B, NUM_WORKERS, NCHUNK, S_CHUNK),
        (1, 2, 0, 3)).reshape(NUM_WORKERS, NCHUNK, ROWS)
    out = _run(idx_re, token_table, pos_table)
    return out.reshape(B, S, EMBED)
